# Initial kernel scaffold; baseline (speedup 1.0000x reference)
import jax
import jax.numpy as jnp
from jax.experimental import pallas as pl


def kernel(x, edge_index, edge_weight, W1, b1, W2, b2, W3, b3, W4, b4):
    raise NotImplementedError("write your pallas kernel here")


# speedup vs baseline: 3.4744x; 3.4744x over previous
import functools

import jax
import jax.numpy as jnp
from jax import lax
from jax.experimental import pallas as pl
from jax.experimental.pallas import tpu as pltpu
from jax.experimental.pallas import tpu_sc as plsc

NC = 2
NS = 16
L = 16
DH = 64


def _spmm_sc(sup2d, src4d, dst3d, w3d, *, n_nodes, epw, k):
    nchunk = epw // k
    npairs = (nchunk - 1) // 2
    rpt = (n_nodes // NS) // 8 * 8
    nj = DH // L

    mesh = plsc.VectorSubcoreMesh(core_axis_name="c", subcore_axis_name="s")

    @functools.partial(
        pl.kernel,
        mesh=mesh,
        compiler_params=pltpu.CompilerParams(use_tc_tiling_on_sc=False),
        out_type=jax.ShapeDtypeStruct((NC, n_nodes, DH), jnp.float32),
        scratch_types=[
            pltpu.VMEM((nchunk, k), jnp.int32),
            pltpu.VMEM((nchunk, k), jnp.int32),
            pltpu.VMEM((1, epw), jnp.float32),
            pltpu.VMEM((2 * k, DH), jnp.float32),
            pltpu.VMEM_SHARED((n_nodes, DH), jnp.float32),
            pltpu.SemaphoreType.DMA,
            pltpu.SemaphoreType.DMA,
        ],
    )
    def spmm(sup_hbm, src_hbm, dst_hbm, w_hbm, out_hbm,
             srcv, dstv, wv, rows, acc, gsem0, gsem1):
        cid = lax.axis_index("c")
        sid = lax.axis_index("s")

        zero = jnp.zeros((L,), jnp.float32)

        def zrow(i, carry):
            for j in range(nj):
                rows[i, pl.ds(j * L, L)] = zero
            return carry

        lax.fori_loop(0, 2 * k, zrow, 0)

        base_r = sid * rpt
        my_rows = n_nodes - (NS - 1) * rpt
        for sel, cnt in ((sid < NS - 1, rpt), (sid == NS - 1, my_rows)):
            @pl.when(sel)
            def _():
                done = 0
                while done < cnt:
                    step = min(2 * k, cnt - done)
                    pltpu.sync_copy(rows.at[pl.ds(0, step)],
                                    acc.at[pl.ds(base_r + done, step)])
                    done += step
        plsc.subcore_barrier()

        pltpu.sync_copy(src_hbm.at[cid, sid], srcv)
        pltpu.sync_copy(dst_hbm.at[sid], dstv)
        pltpu.sync_copy(w_hbm.at[sid], wv)

        def buf(b):
            return rows.at[pl.ds(b * k, k)]

        def sems(b):
            return gsem1 if b else gsem0

        def start_gather(c, b):
            pltpu.async_copy(sup_hbm.at[srcv.at[c]], buf(b), sems(b))

        def wait_gather(c, b):
            pltpu.make_async_copy(sup_hbm.at[srcv.at[c]], buf(b), sems(b)).wait()

        def process(c, b):
            def group(m, carry):
                base_i = m * L
                w16 = wv[0, pl.ds(c * k + base_i, L)]
                for t in range(L):
                    wb = jnp.broadcast_to(w16[t], (L,))
                    i = base_i + t
                    for j in range(nj):
                        sl = pl.ds(j * L, L)
                        rows[b * k + i, sl] = rows[b * k + i, sl] * wb
                return carry

            lax.fori_loop(0, k // L, group, 0)
            pltpu.sync_copy(buf(b), acc.at[dstv.at[c]], add=True)

        start_gather(0, 0)

        def pair(g, carry):
            c0 = 2 * g
            start_gather(c0 + 1, 1)
            wait_gather(c0, 0)
            process(c0, 0)
            start_gather(c0 + 2, 0)
            wait_gather(c0 + 1, 1)
            process(c0 + 1, 1)
            return carry

        lax.fori_loop(0, npairs, pair, 0)
        c_t = 2 * npairs
        if nchunk - c_t == 2:
            start_gather(c_t + 1, 1)
            wait_gather(c_t, 0)
            process(c_t, 0)
            wait_gather(c_t + 1, 1)
            process(c_t + 1, 1)
        else:
            wait_gather(c_t, 0)
            process(c_t, 0)

        plsc.subcore_barrier()
        for sel, cnt in ((sid < NS - 1, rpt), (sid == NS - 1, my_rows)):
            @pl.when(sel)
            def _():
                pltpu.sync_copy(acc.at[pl.ds(base_r, cnt)],
                                out_hbm.at[cid, pl.ds(base_r, cnt)])

    return spmm(sup2d, src4d, dst3d, w3d)


def _split_cols(res, o_ref):
    o_ref[0] = res[:, :DH]
    o_ref[1] = res[:, DH:]


def _mm_first(x, W, n_nodes, blk):
    def body(x_ref, w_ref, o_ref):
        _split_cols(jnp.dot(x_ref[...], w_ref[...],
                            preferred_element_type=jnp.float32), o_ref)

    return pl.pallas_call(
        body,
        grid=(n_nodes // blk,),
        in_specs=[pl.BlockSpec((blk, 128), lambda i: (i, 0)),
                  pl.BlockSpec((128, 128), lambda i: (0, 0))],
        out_specs=pl.BlockSpec((2, blk, DH), lambda i: (0, i, 0)),
        out_shape=jax.ShapeDtypeStruct((2, n_nodes, DH), jnp.float32),
    )(x, W)


def _mm_fused(parts, b, W, n_nodes, blk):
    def body(p_ref, b_ref, w_ref, o_ref):
        agg = jnp.concatenate([p_ref[0], p_ref[1]], axis=1)
        h = jnp.maximum(agg + b_ref[...], 0.0)
        _split_cols(jnp.dot(h, w_ref[...], preferred_element_type=jnp.float32),
                    o_ref)

    return pl.pallas_call(
        body,
        grid=(n_nodes // blk,),
        in_specs=[pl.BlockSpec((2, blk, DH), lambda i: (0, i, 0)),
                  pl.BlockSpec((1, 128), lambda i: (0, 0)),
                  pl.BlockSpec((128, 128), lambda i: (0, 0))],
        out_specs=pl.BlockSpec((2, blk, DH), lambda i: (0, i, 0)),
        out_shape=jax.ShapeDtypeStruct((2, n_nodes, DH), jnp.float32),
    )(parts, b.reshape(1, 128), W)


def _final_sum(parts, b, n_nodes, blk):
    def body(p_ref, b_ref, o_ref):
        o_ref[...] = jnp.concatenate([p_ref[0], p_ref[1]], axis=1) + b_ref[...]

    return pl.pallas_call(
        body,
        grid=(n_nodes // blk,),
        in_specs=[pl.BlockSpec((2, blk, DH), lambda i: (0, i, 0)),
                  pl.BlockSpec((1, 128), lambda i: (0, 0))],
        out_specs=pl.BlockSpec((blk, 128), lambda i: (i, 0)),
        out_shape=jax.ShapeDtypeStruct((n_nodes, 128), jnp.float32),
    )(parts, b.reshape(1, 128))


def kernel(x, edge_index, edge_weight, W1, b1, W2, b2, W3, b3, W4, b4):
    n_nodes = x.shape[0]
    e = edge_weight.shape[0]
    k = 64
    blk = 1000

    epw = e // NS
    epw_pad = -(-epw // k) * k
    pad = epw_pad - epw

    def worker_layout(a, fill):
        a = a.reshape(NS, epw)
        if pad:
            a = jnp.pad(a, ((0, 0), (0, pad)), constant_values=fill)
        return a

    nchunk = epw_pad // k
    dst3d = worker_layout(edge_index[0], 0).reshape(NS, nchunk, k)
    src3d = worker_layout(edge_index[1], 0).reshape(NS, nchunk, k)
    src4d = src3d[None] + (jnp.arange(NC, dtype=jnp.int32) * n_nodes)[:, None, None, None]
    w3d = worker_layout(edge_weight, 0.0).reshape(NS, 1, epw_pad)

    W4p = jnp.pad(W4, ((0, 0), (0, 128 - W4.shape[1])))
    b4p = jnp.pad(b4, (0, 128 - b4.shape[0]))

    spmm = functools.partial(_spmm_sc, n_nodes=n_nodes, epw=epw_pad, k=k)

    s = _mm_first(x, W1, n_nodes, blk)
    p = spmm(s.reshape(NC * n_nodes, DH), src4d, dst3d, w3d)
    s = _mm_fused(p, b1, W2, n_nodes, blk)
    p = spmm(s.reshape(NC * n_nodes, DH), src4d, dst3d, w3d)
    s = _mm_fused(p, b2, W3, n_nodes, blk)
    p = spmm(s.reshape(NC * n_nodes, DH), src4d, dst3d, w3d)
    s = _mm_fused(p, b3, W4p, n_nodes, blk)
    p = spmm(s.reshape(NC * n_nodes, DH), src4d, dst3d, w3d)
    out = _final_sum(p, b4p, n_nodes, blk)
    return out[:, :W4.shape[1]]


# speedup vs baseline: 3.5166x; 1.0122x over previous
import functools

import jax
import jax.numpy as jnp
from jax import lax
from jax.experimental import pallas as pl
from jax.experimental.pallas import tpu as pltpu
from jax.experimental.pallas import tpu_sc as plsc

NC = 2
NS = 16
L = 16
DH = 64


def _spmm_sc(sup2d, src4d, dst3d, w3d, *, n_nodes, epw, k):
    nchunk = epw // k
    assert nchunk % 4 == 0 and nchunk >= 8
    rpt = (n_nodes // NS) // 8 * 8
    nj = DH // L
    nbuf = 4

    mesh = plsc.VectorSubcoreMesh(core_axis_name="c", subcore_axis_name="s")

    @functools.partial(
        pl.kernel,
        mesh=mesh,
        compiler_params=pltpu.CompilerParams(use_tc_tiling_on_sc=False),
        out_type=jax.ShapeDtypeStruct((NC, n_nodes, DH), jnp.float32),
        scratch_types=[
            pltpu.VMEM((nchunk, k), jnp.int32),
            pltpu.VMEM((nchunk, k), jnp.int32),
            pltpu.VMEM((1, epw), jnp.float32),
            pltpu.VMEM((nbuf * k, DH), jnp.float32),
            pltpu.VMEM_SHARED((n_nodes, DH), jnp.float32),
            [pltpu.SemaphoreType.DMA] * nbuf,
            [pltpu.SemaphoreType.DMA] * nbuf,
        ],
    )
    def spmm(sup_hbm, src_hbm, dst_hbm, w_hbm, out_hbm,
             srcv, dstv, wv, rows, acc, gsems, ssems):
        cid = lax.axis_index("c")
        sid = lax.axis_index("s")

        zero = jnp.zeros((L,), jnp.float32)

        def zrow(i, carry):
            for j in range(nj):
                rows[i, pl.ds(j * L, L)] = zero
            return carry

        lax.fori_loop(0, nbuf * k, zrow, 0)

        base_r = sid * rpt
        my_rows = n_nodes - (NS - 1) * rpt
        for sel, cnt in ((sid < NS - 1, rpt), (sid == NS - 1, my_rows)):
            @pl.when(sel)
            def _():
                done = 0
                while done < cnt:
                    step = min(nbuf * k, cnt - done)
                    pltpu.sync_copy(rows.at[pl.ds(0, step)],
                                    acc.at[pl.ds(base_r + done, step)])
                    done += step
        plsc.subcore_barrier()

        pltpu.sync_copy(src_hbm.at[cid, sid], srcv)
        pltpu.sync_copy(dst_hbm.at[sid], dstv)
        pltpu.sync_copy(w_hbm.at[sid], wv)

        def buf(b):
            return rows.at[pl.ds(b * k, k)]

        def start_gather(c, b):
            pltpu.async_copy(sup_hbm.at[srcv.at[c]], buf(b), gsems[b])

        def wait_gather(c, b):
            pltpu.make_async_copy(sup_hbm.at[srcv.at[c]], buf(b), gsems[b]).wait()

        def start_scatter(c, b):
            pltpu.async_copy(buf(b), acc.at[dstv.at[c]], ssems[b], add=True)

        def wait_scatter(c, b):
            pltpu.make_async_copy(buf(b), acc.at[dstv.at[c]], ssems[b]).wait()

        def process(c, b):
            def group(m, carry):
                base_i = m * L
                w16 = wv[0, pl.ds(c * k + base_i, L)]
                for t in range(L):
                    wb = jnp.broadcast_to(w16[t], (L,))
                    i = base_i + t
                    for j in range(nj):
                        sl = pl.ds(j * L, L)
                        rows[b * k + i, sl] = rows[b * k + i, sl] * wb
                return carry

            lax.fori_loop(0, k // L, group, 0)

        def slot(c, b, head, tail):
            if not head:
                wait_scatter(c - 2, (b + 2) % nbuf)
            if not tail:
                start_gather(c + 2, (b + 2) % nbuf)
            wait_gather(c, b)
            process(c, b)
            start_scatter(c, b)

        start_gather(0, 0)
        start_gather(1, 1)
        slot(0, 0, True, False)
        slot(1, 1, True, False)

        def quad(g, carry):
            c0 = 4 * g + 2
            for j in range(4):
                slot(c0 + j, (2 + j) % nbuf, False, False)
            return carry

        lax.fori_loop(0, (nchunk - 4) // 4, quad, 0)
        slot(nchunk - 2, (nchunk - 2) % nbuf, False, True)
        slot(nchunk - 1, (nchunk - 1) % nbuf, False, True)
        wait_scatter(nchunk - 2, (nchunk - 2) % nbuf)
        wait_scatter(nchunk - 1, (nchunk - 1) % nbuf)

        plsc.subcore_barrier()
        for sel, cnt in ((sid < NS - 1, rpt), (sid == NS - 1, my_rows)):
            @pl.when(sel)
            def _():
                pltpu.sync_copy(acc.at[pl.ds(base_r, cnt)],
                                out_hbm.at[cid, pl.ds(base_r, cnt)])

    return spmm(sup2d, src4d, dst3d, w3d)


def _split_cols(res, o_ref):
    o_ref[0] = res[:, :DH]
    o_ref[1] = res[:, DH:]


def _mm_first(x, W, n_nodes, blk):
    def body(x_ref, w_ref, o_ref):
        _split_cols(jnp.dot(x_ref[...], w_ref[...],
                            preferred_element_type=jnp.float32), o_ref)

    return pl.pallas_call(
        body,
        grid=(n_nodes // blk,),
        in_specs=[pl.BlockSpec((blk, 128), lambda i: (i, 0)),
                  pl.BlockSpec((128, 128), lambda i: (0, 0))],
        out_specs=pl.BlockSpec((2, blk, DH), lambda i: (0, i, 0)),
        out_shape=jax.ShapeDtypeStruct((2, n_nodes, DH), jnp.float32),
    )(x, W)


def _mm_fused(parts, b, W, n_nodes, blk):
    def body(p_ref, b_ref, w_ref, o_ref):
        agg = jnp.concatenate([p_ref[0], p_ref[1]], axis=1)
        h = jnp.maximum(agg + b_ref[...], 0.0)
        _split_cols(jnp.dot(h, w_ref[...], preferred_element_type=jnp.float32),
                    o_ref)

    return pl.pallas_call(
        body,
        grid=(n_nodes // blk,),
        in_specs=[pl.BlockSpec((2, blk, DH), lambda i: (0, i, 0)),
                  pl.BlockSpec((1, 128), lambda i: (0, 0)),
                  pl.BlockSpec((128, 128), lambda i: (0, 0))],
        out_specs=pl.BlockSpec((2, blk, DH), lambda i: (0, i, 0)),
        out_shape=jax.ShapeDtypeStruct((2, n_nodes, DH), jnp.float32),
    )(parts, b.reshape(1, 128), W)


def _final_sum(parts, b, n_nodes, blk):
    def body(p_ref, b_ref, o_ref):
        o_ref[...] = jnp.concatenate([p_ref[0], p_ref[1]], axis=1) + b_ref[...]

    return pl.pallas_call(
        body,
        grid=(n_nodes // blk,),
        in_specs=[pl.BlockSpec((2, blk, DH), lambda i: (0, i, 0)),
                  pl.BlockSpec((1, 128), lambda i: (0, 0))],
        out_specs=pl.BlockSpec((blk, 128), lambda i: (i, 0)),
        out_shape=jax.ShapeDtypeStruct((n_nodes, 128), jnp.float32),
    )(parts, b.reshape(1, 128))


def kernel(x, edge_index, edge_weight, W1, b1, W2, b2, W3, b3, W4, b4):
    n_nodes = x.shape[0]
    e = edge_weight.shape[0]
    k = 64
    blk = 1000

    epw = e // NS
    epw_pad = -(-epw // (4 * k)) * (4 * k)
    pad = epw_pad - epw

    def worker_layout(a, fill):
        a = a.reshape(NS, epw)
        if pad:
            a = jnp.pad(a, ((0, 0), (0, pad)), constant_values=fill)
        return a

    nchunk = epw_pad // k
    dst3d = worker_layout(edge_index[0], 0).reshape(NS, nchunk, k)
    src3d = worker_layout(edge_index[1], 0).reshape(NS, nchunk, k)
    src4d = src3d[None] + (jnp.arange(NC, dtype=jnp.int32) * n_nodes)[:, None, None, None]
    w3d = worker_layout(edge_weight, 0.0).reshape(NS, 1, epw_pad)

    W4p = jnp.pad(W4, ((0, 0), (0, 128 - W4.shape[1])))
    b4p = jnp.pad(b4, (0, 128 - b4.shape[0]))

    spmm = functools.partial(_spmm_sc, n_nodes=n_nodes, epw=epw_pad, k=k)

    s = _mm_first(x, W1, n_nodes, blk)
    p = spmm(s.reshape(NC * n_nodes, DH), src4d, dst3d, w3d)
    s = _mm_fused(p, b1, W2, n_nodes, blk)
    p = spmm(s.reshape(NC * n_nodes, DH), src4d, dst3d, w3d)
    s = _mm_fused(p, b2, W3, n_nodes, blk)
    p = spmm(s.reshape(NC * n_nodes, DH), src4d, dst3d, w3d)
    s = _mm_fused(p, b3, W4p, n_nodes, blk)
    p = spmm(s.reshape(NC * n_nodes, DH), src4d, dst3d, w3d)
    out = _final_sum(p, b4p, n_nodes, blk)
    return out[:, :W4.shape[1]]


# speedup vs baseline: 6.8592x; 1.9505x over previous
import functools

import jax
import jax.numpy as jnp
from jax import lax
from jax.experimental import pallas as pl
from jax.experimental.pallas import tpu as pltpu
from jax.experimental.pallas import tpu_sc as plsc

NC = 2
NS = 16
L = 16
DH = 64


def _spmm_sc(sup2d, src4d, dst3d, w3d, *, n_nodes, epw, k):
    nchunk = epw // k
    assert nchunk % 4 == 0 and nchunk >= 8
    rpt = (n_nodes // NS) // 8 * 8
    nj = DH // L
    nbuf = 4

    mesh = plsc.VectorSubcoreMesh(core_axis_name="c", subcore_axis_name="s")

    @functools.partial(
        pl.kernel,
        mesh=mesh,
        compiler_params=pltpu.CompilerParams(use_tc_tiling_on_sc=False),
        out_type=jax.ShapeDtypeStruct((NC, n_nodes, DH), jnp.float32),
        scratch_types=[
            pltpu.VMEM((nchunk, k), jnp.int32),
            pltpu.VMEM((nchunk, k), jnp.int32),
            pltpu.VMEM((1, epw), jnp.float32),
            pltpu.VMEM((nbuf * k, DH), jnp.float32),
            pltpu.VMEM_SHARED((n_nodes, DH), jnp.float32),
            [pltpu.SemaphoreType.DMA] * nbuf,
            [pltpu.SemaphoreType.DMA] * nbuf,
        ],
    )
    def spmm(sup_hbm, src_hbm, dst_hbm, w_hbm, out_hbm,
             srcv, dstv, wv, rows, acc, gsems, ssems):
        cid = lax.axis_index("c")
        sid = lax.axis_index("s")

        zero = jnp.zeros((L,), jnp.float32)

        def zrow(i, carry):
            for j in range(nj):
                rows[i, pl.ds(j * L, L)] = zero
            return carry

        lax.fori_loop(0, nbuf * k, zrow, 0)

        base_r = sid * rpt
        my_rows = n_nodes - (NS - 1) * rpt
        for sel, cnt in ((sid < NS - 1, rpt), (sid == NS - 1, my_rows)):
            @pl.when(sel)
            def _():
                done = 0
                while done < cnt:
                    step = min(nbuf * k, cnt - done)
                    pltpu.sync_copy(rows.at[pl.ds(0, step)],
                                    acc.at[pl.ds(base_r + done, step)])
                    done += step
        plsc.subcore_barrier()

        pltpu.sync_copy(src_hbm.at[cid, sid], srcv)
        pltpu.sync_copy(dst_hbm.at[sid], dstv)
        pltpu.sync_copy(w_hbm.at[sid], wv)

        def buf(b):
            return rows.at[pl.ds(b * k, k)]

        def start_gather(c, b):
            pltpu.async_copy(sup_hbm.at[srcv.at[c]], buf(b), gsems[b])

        def wait_gather(c, b):
            pltpu.make_async_copy(sup_hbm.at[srcv.at[c]], buf(b), gsems[b]).wait()

        def start_scatter(c, b):
            pltpu.async_copy(buf(b), acc.at[dstv.at[c]], ssems[b], add=True)

        def wait_scatter(c, b):
            pltpu.make_async_copy(buf(b), acc.at[dstv.at[c]], ssems[b]).wait()

        def process(c, b):
            @plsc.parallel_loop(0, k // L, 1, unroll=2)
            def _(m):
                base_i = m * L
                w16 = wv[0, pl.ds(c * k + base_i, L)]
                for t in range(L):
                    wb = jnp.broadcast_to(w16[t], (L,))
                    i = base_i + t
                    for j in range(nj):
                        sl = pl.ds(j * L, L)
                        rows[b * k + i, sl] = rows[b * k + i, sl] * wb

        def slot(c, b, head, tail):
            if not head:
                wait_scatter(c - 2, (b + 2) % nbuf)
            if not tail:
                start_gather(c + 2, (b + 2) % nbuf)
            wait_gather(c, b)
            process(c, b)
            start_scatter(c, b)

        start_gather(0, 0)
        start_gather(1, 1)
        slot(0, 0, True, False)
        slot(1, 1, True, False)

        def quad(g, carry):
            c0 = 4 * g + 2
            for j in range(4):
                slot(c0 + j, (2 + j) % nbuf, False, False)
            return carry

        lax.fori_loop(0, (nchunk - 4) // 4, quad, 0)
        slot(nchunk - 2, (nchunk - 2) % nbuf, False, True)
        slot(nchunk - 1, (nchunk - 1) % nbuf, False, True)
        wait_scatter(nchunk - 2, (nchunk - 2) % nbuf)
        wait_scatter(nchunk - 1, (nchunk - 1) % nbuf)

        plsc.subcore_barrier()
        for sel, cnt in ((sid < NS - 1, rpt), (sid == NS - 1, my_rows)):
            @pl.when(sel)
            def _():
                pltpu.sync_copy(acc.at[pl.ds(base_r, cnt)],
                                out_hbm.at[cid, pl.ds(base_r, cnt)])

    return spmm(sup2d, src4d, dst3d, w3d)


def _split_cols(res, o_ref):
    o_ref[0] = res[:, :DH]
    o_ref[1] = res[:, DH:]


def _mm_first(x, W, n_nodes, blk):
    def body(x_ref, w_ref, o_ref):
        _split_cols(jnp.dot(x_ref[...], w_ref[...],
                            preferred_element_type=jnp.float32), o_ref)

    return pl.pallas_call(
        body,
        grid=(n_nodes // blk,),
        in_specs=[pl.BlockSpec((blk, 128), lambda i: (i, 0)),
                  pl.BlockSpec((128, 128), lambda i: (0, 0))],
        out_specs=pl.BlockSpec((2, blk, DH), lambda i: (0, i, 0)),
        out_shape=jax.ShapeDtypeStruct((2, n_nodes, DH), jnp.float32),
    )(x, W)


def _mm_fused(parts, b, W, n_nodes, blk):
    def body(p_ref, b_ref, w_ref, o_ref):
        agg = jnp.concatenate([p_ref[0], p_ref[1]], axis=1)
        h = jnp.maximum(agg + b_ref[...], 0.0)
        _split_cols(jnp.dot(h, w_ref[...], preferred_element_type=jnp.float32),
                    o_ref)

    return pl.pallas_call(
        body,
        grid=(n_nodes // blk,),
        in_specs=[pl.BlockSpec((2, blk, DH), lambda i: (0, i, 0)),
                  pl.BlockSpec((1, 128), lambda i: (0, 0)),
                  pl.BlockSpec((128, 128), lambda i: (0, 0))],
        out_specs=pl.BlockSpec((2, blk, DH), lambda i: (0, i, 0)),
        out_shape=jax.ShapeDtypeStruct((2, n_nodes, DH), jnp.float32),
    )(parts, b.reshape(1, 128), W)


def _final_sum(parts, b, n_nodes, blk):
    def body(p_ref, b_ref, o_ref):
        o_ref[...] = jnp.concatenate([p_ref[0], p_ref[1]], axis=1) + b_ref[...]

    return pl.pallas_call(
        body,
        grid=(n_nodes // blk,),
        in_specs=[pl.BlockSpec((2, blk, DH), lambda i: (0, i, 0)),
                  pl.BlockSpec((1, 128), lambda i: (0, 0))],
        out_specs=pl.BlockSpec((blk, 128), lambda i: (i, 0)),
        out_shape=jax.ShapeDtypeStruct((n_nodes, 128), jnp.float32),
    )(parts, b.reshape(1, 128))


def kernel(x, edge_index, edge_weight, W1, b1, W2, b2, W3, b3, W4, b4):
    n_nodes = x.shape[0]
    e = edge_weight.shape[0]
    k = 64
    blk = 1000

    epw = e // NS
    epw_pad = -(-epw // (4 * k)) * (4 * k)
    pad = epw_pad - epw

    def worker_layout(a, fill):
        a = a.reshape(NS, epw)
        if pad:
            a = jnp.pad(a, ((0, 0), (0, pad)), constant_values=fill)
        return a

    nchunk = epw_pad // k
    dst3d = worker_layout(edge_index[0], 0).reshape(NS, nchunk, k)
    src3d = worker_layout(edge_index[1], 0).reshape(NS, nchunk, k)
    src4d = src3d[None] + (jnp.arange(NC, dtype=jnp.int32) * n_nodes)[:, None, None, None]
    w3d = worker_layout(edge_weight, 0.0).reshape(NS, 1, epw_pad)

    W4p = jnp.pad(W4, ((0, 0), (0, 128 - W4.shape[1])))
    b4p = jnp.pad(b4, (0, 128 - b4.shape[0]))

    spmm = functools.partial(_spmm_sc, n_nodes=n_nodes, epw=epw_pad, k=k)

    s = _mm_first(x, W1, n_nodes, blk)
    p = spmm(s.reshape(NC * n_nodes, DH), src4d, dst3d, w3d)
    s = _mm_fused(p, b1, W2, n_nodes, blk)
    p = spmm(s.reshape(NC * n_nodes, DH), src4d, dst3d, w3d)
    s = _mm_fused(p, b2, W3, n_nodes, blk)
    p = spmm(s.reshape(NC * n_nodes, DH), src4d, dst3d, w3d)
    s = _mm_fused(p, b3, W4p, n_nodes, blk)
    p = spmm(s.reshape(NC * n_nodes, DH), src4d, dst3d, w3d)
    out = _final_sum(p, b4p, n_nodes, blk)
    return out[:, :W4.shape[1]]


# speedup vs baseline: 7.3157x; 1.0666x over previous
import functools

import jax
import jax.numpy as jnp
import numpy as np
from jax import lax
from jax.experimental import pallas as pl
from jax.experimental.pallas import tpu as pltpu
from jax.experimental.pallas import tpu_sc as plsc

NC = 2
NS = 16
L = 16
DH = 64
_PERMH = np.concatenate([np.arange(0, 32, 2), np.arange(1, 32, 2),
                         np.arange(32, 64, 2), np.arange(33, 64, 2)])
_SH = np.argsort(_PERMH)
_COLPERM = np.concatenate([_SH, _SH + 64])


def _spmm_sc(sup2d, src4d, dst3d, w3d, *, n_nodes, epw, k):
    nchunk = epw // k
    assert nchunk % 4 == 0 and nchunk >= 8
    rpt = (n_nodes // NS) // 8 * 8
    nj = DH // L
    nbuf = 4

    mesh = plsc.VectorSubcoreMesh(core_axis_name="c", subcore_axis_name="s")

    @functools.partial(
        pl.kernel,
        mesh=mesh,
        compiler_params=pltpu.CompilerParams(use_tc_tiling_on_sc=False,
                                             needs_layout_passes=False),
        out_type=jax.ShapeDtypeStruct((NC, n_nodes, DH), jnp.float32),
        scratch_types=[
            pltpu.VMEM((nchunk, k), jnp.int32),
            pltpu.VMEM((nchunk, k), jnp.int32),
            pltpu.VMEM((1, epw), jnp.float32),
            pltpu.VMEM((nbuf * k, DH), jnp.bfloat16),
            pltpu.VMEM((nbuf * k, DH), jnp.float32),
            pltpu.VMEM_SHARED((n_nodes, DH), jnp.float32),
            [pltpu.SemaphoreType.DMA] * nbuf,
            [pltpu.SemaphoreType.DMA] * nbuf,
        ],
    )
    def spmm(sup_hbm, src_hbm, dst_hbm, w_hbm, out_hbm,
             srcv, dstv, wv, gbuf, sbuf, acc, gsems, ssems):
        cid = lax.axis_index("c")
        sid = lax.axis_index("s")

        zero = jnp.zeros((L,), jnp.float32)

        def zrow(i, carry):
            for j in range(nj):
                sbuf[i, pl.ds(j * L, L)] = zero
            return carry

        lax.fori_loop(0, nbuf * k, zrow, 0)

        base_r = sid * rpt
        my_rows = n_nodes - (NS - 1) * rpt
        for sel, cnt in ((sid < NS - 1, rpt), (sid == NS - 1, my_rows)):
            @pl.when(sel)
            def _():
                done = 0
                while done < cnt:
                    step = min(nbuf * k, cnt - done)
                    pltpu.sync_copy(sbuf.at[pl.ds(0, step)],
                                    acc.at[pl.ds(base_r + done, step)])
                    done += step
        plsc.subcore_barrier()

        pltpu.sync_copy(src_hbm.at[cid, sid], srcv)
        pltpu.sync_copy(dst_hbm.at[sid], dstv)
        pltpu.sync_copy(w_hbm.at[sid], wv)

        def gslice(b):
            return gbuf.at[pl.ds(b * k, k)]

        def sslice(b):
            return sbuf.at[pl.ds(b * k, k)]

        def start_gather(c, b):
            pltpu.async_copy(sup_hbm.at[srcv.at[c]], gslice(b), gsems[b])

        def wait_gather(c, b):
            pltpu.make_async_copy(sup_hbm.at[srcv.at[c]], gslice(b),
                                  gsems[b]).wait()

        def start_scatter(c, b):
            pltpu.async_copy(sslice(b), acc.at[dstv.at[c]], ssems[b], add=True)

        def wait_scatter(c, b):
            pltpu.make_async_copy(sslice(b), acc.at[dstv.at[c]],
                                  ssems[b]).wait()

        def process(c, b):
            @plsc.parallel_loop(0, k // L, 1, unroll=2)
            def _(m):
                base_i = m * L
                w16 = wv[0, pl.ds(c * k + base_i, L)]
                for t in range(L):
                    wb = jnp.broadcast_to(w16[t], (L,))
                    i = base_i + t
                    for j in range(DH // (2 * L)):
                        v = gbuf[b * k + i, pl.ds(j * 2 * L, 2 * L)]
                        ev, od = plsc.unpack(
                            v, format=plsc.PackFormat.INTERLEAVED,
                            preferred_element_type=jnp.float32)
                        sbuf[b * k + i, pl.ds(2 * j * L, L)] = ev * wb
                        sbuf[b * k + i, pl.ds((2 * j + 1) * L, L)] = od * wb

        def slot(c, b, head, tail):
            if not head:
                wait_scatter(c - 2, (b + 2) % nbuf)
            if not tail:
                start_gather(c + 2, (b + 2) % nbuf)
            wait_gather(c, b)
            process(c, b)
            start_scatter(c, b)

        start_gather(0, 0)
        start_gather(1, 1)
        slot(0, 0, True, False)
        slot(1, 1, True, False)

        def quad(g, carry):
            c0 = 4 * g + 2
            for j in range(4):
                slot(c0 + j, (2 + j) % nbuf, False, False)
            return carry

        lax.fori_loop(0, (nchunk - 4) // 4, quad, 0)
        slot(nchunk - 2, (nchunk - 2) % nbuf, False, True)
        slot(nchunk - 1, (nchunk - 1) % nbuf, False, True)
        wait_scatter(nchunk - 2, (nchunk - 2) % nbuf)
        wait_scatter(nchunk - 1, (nchunk - 1) % nbuf)

        plsc.subcore_barrier()
        for sel, cnt in ((sid < NS - 1, rpt), (sid == NS - 1, my_rows)):
            @pl.when(sel)
            def _():
                pltpu.sync_copy(acc.at[pl.ds(base_r, cnt)],
                                out_hbm.at[cid, pl.ds(base_r, cnt)])

    return spmm(sup2d, src4d, dst3d, w3d)


def _split_cols(res, o_ref):
    rb = res.astype(jnp.bfloat16)
    o_ref[0] = rb[:, :DH]
    o_ref[1] = rb[:, DH:]


def _mm_first(x, W, n_nodes, blk):
    def body(x_ref, w_ref, o_ref):
        _split_cols(jnp.dot(x_ref[...], w_ref[...],
                            preferred_element_type=jnp.float32), o_ref)

    return pl.pallas_call(
        body,
        grid=(n_nodes // blk,),
        in_specs=[pl.BlockSpec((blk, 128), lambda i: (i, 0)),
                  pl.BlockSpec((128, 128), lambda i: (0, 0))],
        out_specs=pl.BlockSpec((2, blk, DH), lambda i: (0, i, 0)),
        out_shape=jax.ShapeDtypeStruct((2, n_nodes, DH), jnp.bfloat16),
    )(x, W)


def _mm_fused(parts, b, W, n_nodes, blk):
    def body(p_ref, b_ref, w_ref, o_ref):
        agg = jnp.concatenate([p_ref[0], p_ref[1]], axis=1)
        h = jnp.maximum(agg + b_ref[...], 0.0)
        _split_cols(jnp.dot(h, w_ref[...], preferred_element_type=jnp.float32),
                    o_ref)

    return pl.pallas_call(
        body,
        grid=(n_nodes // blk,),
        in_specs=[pl.BlockSpec((2, blk, DH), lambda i: (0, i, 0)),
                  pl.BlockSpec((1, 128), lambda i: (0, 0)),
                  pl.BlockSpec((128, 128), lambda i: (0, 0))],
        out_specs=pl.BlockSpec((2, blk, DH), lambda i: (0, i, 0)),
        out_shape=jax.ShapeDtypeStruct((2, n_nodes, DH), jnp.bfloat16),
    )(parts, b.reshape(1, 128), W)


def _final_sum(parts, b, n_nodes, blk):
    def body(p_ref, b_ref, o_ref):
        o_ref[...] = jnp.concatenate([p_ref[0], p_ref[1]], axis=1) + b_ref[...]

    return pl.pallas_call(
        body,
        grid=(n_nodes // blk,),
        in_specs=[pl.BlockSpec((2, blk, DH), lambda i: (0, i, 0)),
                  pl.BlockSpec((1, 128), lambda i: (0, 0))],
        out_specs=pl.BlockSpec((blk, 128), lambda i: (i, 0)),
        out_shape=jax.ShapeDtypeStruct((n_nodes, 128), jnp.float32),
    )(parts, b.reshape(1, 128))


def kernel(x, edge_index, edge_weight, W1, b1, W2, b2, W3, b3, W4, b4):
    n_nodes = x.shape[0]
    e = edge_weight.shape[0]
    k = 64
    blk = 1000

    epw = e // NS
    epw_pad = -(-epw // (4 * k)) * (4 * k)
    pad = epw_pad - epw

    def worker_layout(a, fill):
        a = a.reshape(NS, epw)
        if pad:
            a = jnp.pad(a, ((0, 0), (0, pad)), constant_values=fill)
        return a

    nchunk = epw_pad // k
    dst3d = worker_layout(edge_index[0], 0).reshape(NS, nchunk, k)
    src3d = worker_layout(edge_index[1], 0).reshape(NS, nchunk, k)
    src4d = src3d[None] + (jnp.arange(NC, dtype=jnp.int32) * n_nodes)[:, None, None, None]
    w3d = worker_layout(edge_weight, 0.0).reshape(NS, 1, epw_pad)

    W4p = jnp.pad(W4, ((0, 0), (0, 128 - W4.shape[1])))
    b4p = jnp.pad(b4, (0, 128 - b4.shape[0]))
    W1p, W2p, W3p, W4p = (W[:, _COLPERM] for W in (W1, W2, W3, W4p))

    spmm = functools.partial(_spmm_sc, n_nodes=n_nodes, epw=epw_pad, k=k)

    s = _mm_first(x, W1p, n_nodes, blk)
    p = spmm(s.reshape(NC * n_nodes, DH), src4d, dst3d, w3d)
    s = _mm_fused(p, b1, W2p, n_nodes, blk)
    p = spmm(s.reshape(NC * n_nodes, DH), src4d, dst3d, w3d)
    s = _mm_fused(p, b2, W3p, n_nodes, blk)
    p = spmm(s.reshape(NC * n_nodes, DH), src4d, dst3d, w3d)
    s = _mm_fused(p, b3, W4p, n_nodes, blk)
    p = spmm(s.reshape(NC * n_nodes, DH), src4d, dst3d, w3d)
    out = _final_sum(p, b4p, n_nodes, blk)
    return out[:, :W4.shape[1]]


# speedup vs baseline: 7.3552x; 1.0054x over previous
import functools

import jax
import jax.numpy as jnp
import numpy as np
from jax import lax
from jax.experimental import pallas as pl
from jax.experimental.pallas import tpu as pltpu
from jax.experimental.pallas import tpu_sc as plsc

NC = 2
NS = 16
L = 16
DH = 64
_PERMH = np.concatenate([np.arange(0, 32, 2), np.arange(1, 32, 2),
                         np.arange(32, 64, 2), np.arange(33, 64, 2)])
_SH = np.argsort(_PERMH)
_COLPERM = np.concatenate([_SH, _SH + 64])


def _spmm_sc(sup2d, src4d, dst3d, w3d, b2d, *, n_nodes, epw, k):
    nchunk = epw // k
    assert nchunk % 4 == 0 and nchunk >= 8
    rpt = (n_nodes // NS) // 8 * 8
    nj = DH // L
    nbuf = 4

    mesh = plsc.VectorSubcoreMesh(core_axis_name="c", subcore_axis_name="s")

    @functools.partial(
        pl.kernel,
        mesh=mesh,
        compiler_params=pltpu.CompilerParams(use_tc_tiling_on_sc=False,
                                             needs_layout_passes=False),
        out_type=jax.ShapeDtypeStruct((NC, n_nodes, DH), jnp.float32),
        scratch_types=[
            pltpu.VMEM((nchunk, k), jnp.int32),
            pltpu.VMEM((nchunk, k), jnp.int32),
            pltpu.VMEM((1, epw), jnp.float32),
            pltpu.VMEM((1, DH), jnp.float32),
            pltpu.VMEM((nbuf * k, DH), jnp.bfloat16),
            pltpu.VMEM((nbuf * k, DH), jnp.float32),
            pltpu.VMEM_SHARED((n_nodes, DH), jnp.float32),
            [pltpu.SemaphoreType.DMA] * nbuf,
            [pltpu.SemaphoreType.DMA] * nbuf,
        ],
    )
    def spmm(sup_hbm, src_hbm, dst_hbm, w_hbm, b_hbm, out_hbm,
             srcv, dstv, wv, bv, gbuf, sbuf, acc, gsems, ssems):
        cid = lax.axis_index("c")
        sid = lax.axis_index("s")

        pltpu.sync_copy(b_hbm.at[cid], bv)
        bvec = [bv[0, pl.ds(j * L, L)] for j in range(nj)]

        def brow(i, carry):
            for j in range(nj):
                sbuf[i, pl.ds(j * L, L)] = bvec[j]
            return carry

        lax.fori_loop(0, nbuf * k, brow, 0)

        base_r = sid * rpt
        my_rows = n_nodes - (NS - 1) * rpt
        for sel, cnt in ((sid < NS - 1, rpt), (sid == NS - 1, my_rows)):
            @pl.when(sel)
            def _():
                done = 0
                while done < cnt:
                    step = min(nbuf * k, cnt - done)
                    pltpu.sync_copy(sbuf.at[pl.ds(0, step)],
                                    acc.at[pl.ds(base_r + done, step)])
                    done += step
        plsc.subcore_barrier()

        pltpu.sync_copy(src_hbm.at[cid, sid], srcv)
        pltpu.sync_copy(dst_hbm.at[sid], dstv)
        pltpu.sync_copy(w_hbm.at[sid], wv)

        def gslice(b):
            return gbuf.at[pl.ds(b * k, k)]

        def sslice(b):
            return sbuf.at[pl.ds(b * k, k)]

        def start_gather(c, b):
            pltpu.async_copy(sup_hbm.at[srcv.at[c]], gslice(b), gsems[b])

        def wait_gather(c, b):
            pltpu.make_async_copy(sup_hbm.at[srcv.at[c]], gslice(b),
                                  gsems[b]).wait()

        def start_scatter(c, b):
            pltpu.async_copy(sslice(b), acc.at[dstv.at[c]], ssems[b], add=True)

        def wait_scatter(c, b):
            pltpu.make_async_copy(sslice(b), acc.at[dstv.at[c]],
                                  ssems[b]).wait()

        def process(c, b):
            @plsc.parallel_loop(0, k // L, 1, unroll=2)
            def _(m):
                base_i = m * L
                w16 = wv[0, pl.ds(c * k + base_i, L)]
                for t in range(L):
                    wb = jnp.broadcast_to(w16[t], (L,))
                    i = base_i + t
                    for j in range(DH // (2 * L)):
                        v = gbuf[b * k + i, pl.ds(j * 2 * L, 2 * L)]
                        ev, od = plsc.unpack(
                            v, format=plsc.PackFormat.INTERLEAVED,
                            preferred_element_type=jnp.float32)
                        sbuf[b * k + i, pl.ds(2 * j * L, L)] = ev * wb
                        sbuf[b * k + i, pl.ds((2 * j + 1) * L, L)] = od * wb

        def slot(c, b, head, tail):
            if not head:
                wait_scatter(c - 2, (b + 2) % nbuf)
            if not tail:
                start_gather(c + 2, (b + 2) % nbuf)
            wait_gather(c, b)
            process(c, b)
            start_scatter(c, b)

        start_gather(0, 0)
        start_gather(1, 1)
        slot(0, 0, True, False)
        slot(1, 1, True, False)

        def quad(g, carry):
            c0 = 4 * g + 2
            for j in range(4):
                slot(c0 + j, (2 + j) % nbuf, False, False)
            return carry

        lax.fori_loop(0, (nchunk - 4) // 4, quad, 0)
        slot(nchunk - 2, (nchunk - 2) % nbuf, False, True)
        slot(nchunk - 1, (nchunk - 1) % nbuf, False, True)
        wait_scatter(nchunk - 2, (nchunk - 2) % nbuf)
        wait_scatter(nchunk - 1, (nchunk - 1) % nbuf)

        plsc.subcore_barrier()
        for sel, cnt in ((sid < NS - 1, rpt), (sid == NS - 1, my_rows)):
            @pl.when(sel)
            def _():
                pltpu.sync_copy(acc.at[pl.ds(base_r, cnt)],
                                out_hbm.at[cid, pl.ds(base_r, cnt)])

    return spmm(sup2d, src4d, dst3d, w3d, b2d)


def _split_cols(res, o_ref):
    rb = res.astype(jnp.bfloat16)
    o_ref[0] = rb[:, :DH]
    o_ref[1] = rb[:, DH:]


def _mm_first(x, W, n_nodes, blk):
    def body(x_ref, w_ref, o_ref):
        _split_cols(jnp.dot(x_ref[...], w_ref[...],
                            preferred_element_type=jnp.float32), o_ref)

    return pl.pallas_call(
        body,
        grid=(n_nodes // blk,),
        in_specs=[pl.BlockSpec((blk, 128), lambda i: (i, 0)),
                  pl.BlockSpec((128, 128), lambda i: (0, 0))],
        out_specs=pl.BlockSpec((2, blk, DH), lambda i: (0, i, 0)),
        out_shape=jax.ShapeDtypeStruct((2, n_nodes, DH), jnp.bfloat16),
    )(x, W)


def _mm_fused(parts, W, n_nodes, blk):
    def body(p_ref, w_ref, o_ref):
        h = jnp.maximum(jnp.concatenate([p_ref[0], p_ref[1]], axis=1), 0.0)
        _split_cols(jnp.dot(h, w_ref[...], preferred_element_type=jnp.float32),
                    o_ref)

    return pl.pallas_call(
        body,
        grid=(n_nodes // blk,),
        in_specs=[pl.BlockSpec((2, blk, DH), lambda i: (0, i, 0)),
                  pl.BlockSpec((128, 128), lambda i: (0, 0))],
        out_specs=pl.BlockSpec((2, blk, DH), lambda i: (0, i, 0)),
        out_shape=jax.ShapeDtypeStruct((2, n_nodes, DH), jnp.bfloat16),
    )(parts, W)


def kernel(x, edge_index, edge_weight, W1, b1, W2, b2, W3, b3, W4, b4):
    n_nodes = x.shape[0]
    e = edge_weight.shape[0]
    k = 64
    blk = 1000

    epw = e // NS
    epw_pad = -(-epw // (4 * k)) * (4 * k)
    pad = epw_pad - epw

    def worker_layout(a, fill):
        a = a.reshape(NS, epw)
        if pad:
            a = jnp.pad(a, ((0, 0), (0, pad)), constant_values=fill)
        return a

    nchunk = epw_pad // k
    dst3d = worker_layout(edge_index[0], 0).reshape(NS, nchunk, k)
    src3d = worker_layout(edge_index[1], 0).reshape(NS, nchunk, k)
    src4d = src3d[None] + (jnp.arange(NC, dtype=jnp.int32) * n_nodes)[:, None, None, None]
    w3d = worker_layout(edge_weight, 0.0).reshape(NS, 1, epw_pad)

    W4p = jnp.pad(W4, ((0, 0), (0, 128 - W4.shape[1])))
    b4p = jnp.pad(b4, (0, 128 - b4.shape[0]))
    W1p, W2p, W3p, W4p = (W[:, _COLPERM] for W in (W1, W2, W3, W4p))

    spmm = functools.partial(_spmm_sc, n_nodes=n_nodes, epw=epw_pad, k=k)

    s = _mm_first(x, W1p, n_nodes, blk)
    p = spmm(s.reshape(NC * n_nodes, DH), src4d, dst3d, w3d, b1.reshape(NC, 1, DH))
    s = _mm_fused(p, W2p, n_nodes, blk)
    p = spmm(s.reshape(NC * n_nodes, DH), src4d, dst3d, w3d, b2.reshape(NC, 1, DH))
    s = _mm_fused(p, W3p, n_nodes, blk)
    p = spmm(s.reshape(NC * n_nodes, DH), src4d, dst3d, w3d, b3.reshape(NC, 1, DH))
    s = _mm_fused(p, W4p, n_nodes, blk)
    p = spmm(s.reshape(NC * n_nodes, DH), src4d, dst3d, w3d, b4p.reshape(NC, 1, DH))
    out = jnp.concatenate([p[0], p[1]], axis=1)
    return out[:, :W4.shape[1]]


# speedup vs baseline: 7.4878x; 1.0180x over previous
import functools

import jax
import jax.numpy as jnp
import numpy as np
from jax import lax
from jax.experimental import pallas as pl
from jax.experimental.pallas import tpu as pltpu
from jax.experimental.pallas import tpu_sc as plsc

NC = 2
NS = 16
L = 16
DH = 64
_PERMH = np.concatenate([np.arange(0, 32, 2), np.arange(1, 32, 2),
                         np.arange(32, 64, 2), np.arange(33, 64, 2)])
_SH = np.argsort(_PERMH)
_COLPERM = np.concatenate([_SH, _SH + 64])


def _spmm_sc(sup2d, src4d, dst3d, w3d, b2d, *, n_nodes, epw, k):
    nchunk = epw // k
    assert nchunk % 4 == 0 and nchunk >= 8
    rpt = (n_nodes // NS) // 8 * 8
    nj = DH // L
    nbuf = 4

    mesh = plsc.VectorSubcoreMesh(core_axis_name="c", subcore_axis_name="s")

    @functools.partial(
        pl.kernel,
        mesh=mesh,
        compiler_params=pltpu.CompilerParams(use_tc_tiling_on_sc=False,
                                             needs_layout_passes=False),
        out_type=jax.ShapeDtypeStruct((NC, n_nodes, DH), jnp.float32),
        scratch_types=[
            pltpu.VMEM((nchunk, k), jnp.int32),
            pltpu.VMEM((nchunk, k), jnp.int32),
            pltpu.VMEM((1, epw), jnp.float32),
            pltpu.VMEM((1, DH), jnp.float32),
            pltpu.VMEM((nbuf * k, DH), jnp.bfloat16),
            pltpu.VMEM((nbuf * k, DH), jnp.float32),
            pltpu.VMEM_SHARED((n_nodes, DH), jnp.float32),
            [pltpu.SemaphoreType.DMA] * nbuf,
            [pltpu.SemaphoreType.DMA] * nbuf,
            [pltpu.SemaphoreType.DMA] * 3,
        ],
    )
    def spmm(sup_hbm, src_hbm, dst_hbm, w_hbm, b_hbm, out_hbm,
             srcv, dstv, wv, bv, gbuf, sbuf, acc, gsems, ssems, stsems):
        cid = lax.axis_index("c")
        sid = lax.axis_index("s")

        pltpu.async_copy(src_hbm.at[cid, sid], srcv, stsems[0])
        pltpu.async_copy(dst_hbm.at[sid], dstv, stsems[1])
        pltpu.async_copy(w_hbm.at[sid], wv, stsems[2])

        pltpu.sync_copy(b_hbm.at[cid], bv)
        bvec = [bv[0, pl.ds(j * L, L)] for j in range(nj)]

        def brow(i, carry):
            for j in range(nj):
                sbuf[i, pl.ds(j * L, L)] = bvec[j]
            return carry

        lax.fori_loop(0, nbuf * k, brow, 0)

        pltpu.make_async_copy(src_hbm.at[cid, sid], srcv, stsems[0]).wait()

        base_r = sid * rpt
        my_rows = n_nodes - (NS - 1) * rpt
        for sel, cnt in ((sid < NS - 1, rpt), (sid == NS - 1, my_rows)):
            @pl.when(sel)
            def _():
                done = 0
                while done < cnt:
                    step = min(nbuf * k, cnt - done)
                    pltpu.sync_copy(sbuf.at[pl.ds(0, step)],
                                    acc.at[pl.ds(base_r + done, step)])
                    done += step
        pltpu.make_async_copy(dst_hbm.at[sid], dstv, stsems[1]).wait()
        pltpu.make_async_copy(w_hbm.at[sid], wv, stsems[2]).wait()

        def gslice(b):
            return gbuf.at[pl.ds(b * k, k)]

        def sslice(b):
            return sbuf.at[pl.ds(b * k, k)]

        def start_gather(c, b):
            pltpu.async_copy(sup_hbm.at[srcv.at[c]], gslice(b), gsems[b])

        def wait_gather(c, b):
            pltpu.make_async_copy(sup_hbm.at[srcv.at[c]], gslice(b),
                                  gsems[b]).wait()

        def start_scatter(c, b):
            pltpu.async_copy(sslice(b), acc.at[dstv.at[c]], ssems[b], add=True)

        def wait_scatter(c, b):
            pltpu.make_async_copy(sslice(b), acc.at[dstv.at[c]],
                                  ssems[b]).wait()

        def process(c, b):
            @plsc.parallel_loop(0, k // L, 1, unroll=2)
            def _(m):
                base_i = m * L
                w16 = wv[0, pl.ds(c * k + base_i, L)]
                for t in range(L):
                    wb = jnp.broadcast_to(w16[t], (L,))
                    i = base_i + t
                    for j in range(DH // (2 * L)):
                        v = gbuf[b * k + i, pl.ds(j * 2 * L, 2 * L)]
                        ev, od = plsc.unpack(
                            v, format=plsc.PackFormat.INTERLEAVED,
                            preferred_element_type=jnp.float32)
                        sbuf[b * k + i, pl.ds(2 * j * L, L)] = ev * wb
                        sbuf[b * k + i, pl.ds((2 * j + 1) * L, L)] = od * wb

        def slot(c, b, head, tail):
            if not head:
                wait_scatter(c - 2, (b + 2) % nbuf)
            if not tail:
                start_gather(c + 2, (b + 2) % nbuf)
            wait_gather(c, b)
            process(c, b)
            start_scatter(c, b)

        start_gather(0, 0)
        start_gather(1, 1)
        plsc.subcore_barrier()
        slot(0, 0, True, False)
        slot(1, 1, True, False)

        def quad(g, carry):
            c0 = 4 * g + 2
            for j in range(4):
                slot(c0 + j, (2 + j) % nbuf, False, False)
            return carry

        lax.fori_loop(0, (nchunk - 4) // 4, quad, 0)
        slot(nchunk - 2, (nchunk - 2) % nbuf, False, True)
        slot(nchunk - 1, (nchunk - 1) % nbuf, False, True)
        wait_scatter(nchunk - 2, (nchunk - 2) % nbuf)
        wait_scatter(nchunk - 1, (nchunk - 1) % nbuf)

        plsc.subcore_barrier()
        for sel, cnt in ((sid < NS - 1, rpt), (sid == NS - 1, my_rows)):
            @pl.when(sel)
            def _():
                pltpu.sync_copy(acc.at[pl.ds(base_r, cnt)],
                                out_hbm.at[cid, pl.ds(base_r, cnt)])

    return spmm(sup2d, src4d, dst3d, w3d, b2d)


def _split_cols(res, o_ref):
    rb = res.astype(jnp.bfloat16)
    o_ref[0] = rb[:, :DH]
    o_ref[1] = rb[:, DH:]


def _mm_first(x, W, n_nodes, blk):
    def body(x_ref, w_ref, o_ref):
        _split_cols(jnp.dot(x_ref[...], w_ref[...],
                            preferred_element_type=jnp.float32), o_ref)

    return pl.pallas_call(
        body,
        grid=(n_nodes // blk,),
        in_specs=[pl.BlockSpec((blk, 128), lambda i: (i, 0)),
                  pl.BlockSpec((128, 128), lambda i: (0, 0))],
        out_specs=pl.BlockSpec((2, blk, DH), lambda i: (0, i, 0)),
        out_shape=jax.ShapeDtypeStruct((2, n_nodes, DH), jnp.bfloat16),
    )(x, W)


def _mm_fused(parts, W, n_nodes, blk):
    def body(p_ref, w_ref, o_ref):
        h = jnp.maximum(jnp.concatenate([p_ref[0], p_ref[1]], axis=1), 0.0)
        _split_cols(jnp.dot(h, w_ref[...], preferred_element_type=jnp.float32),
                    o_ref)

    return pl.pallas_call(
        body,
        grid=(n_nodes // blk,),
        in_specs=[pl.BlockSpec((2, blk, DH), lambda i: (0, i, 0)),
                  pl.BlockSpec((128, 128), lambda i: (0, 0))],
        out_specs=pl.BlockSpec((2, blk, DH), lambda i: (0, i, 0)),
        out_shape=jax.ShapeDtypeStruct((2, n_nodes, DH), jnp.bfloat16),
    )(parts, W)


def kernel(x, edge_index, edge_weight, W1, b1, W2, b2, W3, b3, W4, b4):
    n_nodes = x.shape[0]
    e = edge_weight.shape[0]
    k = 64
    blk = 1000

    epw = e // NS
    epw_pad = -(-epw // (4 * k)) * (4 * k)
    pad = epw_pad - epw

    def worker_layout(a, fill):
        a = a.reshape(NS, epw)
        if pad:
            a = jnp.pad(a, ((0, 0), (0, pad)), constant_values=fill)
        return a

    nchunk = epw_pad // k
    dst3d = worker_layout(edge_index[0], 0).reshape(NS, nchunk, k)
    src3d = worker_layout(edge_index[1], 0).reshape(NS, nchunk, k)
    src4d = src3d[None] + (jnp.arange(NC, dtype=jnp.int32) * n_nodes)[:, None, None, None]
    w3d = worker_layout(edge_weight, 0.0).reshape(NS, 1, epw_pad)

    W4p = jnp.pad(W4, ((0, 0), (0, 128 - W4.shape[1])))
    b4p = jnp.pad(b4, (0, 128 - b4.shape[0]))
    W1p, W2p, W3p, W4p = (W[:, _COLPERM] for W in (W1, W2, W3, W4p))

    spmm = functools.partial(_spmm_sc, n_nodes=n_nodes, epw=epw_pad, k=k)

    s = _mm_first(x, W1p, n_nodes, blk)
    p = spmm(s.reshape(NC * n_nodes, DH), src4d, dst3d, w3d, b1.reshape(NC, 1, DH))
    s = _mm_fused(p, W2p, n_nodes, blk)
    p = spmm(s.reshape(NC * n_nodes, DH), src4d, dst3d, w3d, b2.reshape(NC, 1, DH))
    s = _mm_fused(p, W3p, n_nodes, blk)
    p = spmm(s.reshape(NC * n_nodes, DH), src4d, dst3d, w3d, b3.reshape(NC, 1, DH))
    s = _mm_fused(p, W4p, n_nodes, blk)
    p = spmm(s.reshape(NC * n_nodes, DH), src4d, dst3d, w3d, b4p.reshape(NC, 1, DH))
    out = jnp.concatenate([p[0], p[1]], axis=1)
    return out[:, :W4.shape[1]]


# speedup vs baseline: 8.7337x; 1.1664x over previous
import functools

import jax
import jax.numpy as jnp
import numpy as np
from jax import lax
from jax.experimental import pallas as pl
from jax.experimental.pallas import tpu as pltpu
from jax.experimental.pallas import tpu_sc as plsc

NC = 2
NS = 16
L = 16
DH = 64
_PERMH = np.concatenate([np.arange(0, 32, 2), np.arange(1, 32, 2),
                         np.arange(32, 64, 2), np.arange(33, 64, 2)])
_SH = np.argsort(_PERMH)
_COLPERM = np.concatenate([_SH, _SH + 64])


def _spmm_sc(sup2d, src4d, dst3d, w3d, b2d, *, n_nodes, epw, k):
    nchunk = epw // k
    assert nchunk % 4 == 0 and nchunk >= 8
    rpt = (n_nodes // NS) // 8 * 8
    nj = DH // L
    nbuf = 4

    mesh = plsc.VectorSubcoreMesh(core_axis_name="c", subcore_axis_name="s")

    @functools.partial(
        pl.kernel,
        mesh=mesh,
        compiler_params=pltpu.CompilerParams(use_tc_tiling_on_sc=False,
                                             needs_layout_passes=False),
        out_type=jax.ShapeDtypeStruct((NC, n_nodes, DH), jnp.float32),
        scratch_types=[
            pltpu.VMEM((nchunk, k), jnp.int32),
            pltpu.VMEM((nchunk, k), jnp.int32),
            pltpu.VMEM((1, epw), jnp.float32),
            pltpu.VMEM((1, DH), jnp.float32),
            pltpu.VMEM((nbuf * k, DH), jnp.bfloat16),
            pltpu.VMEM((nbuf * k, DH), jnp.float32),
            pltpu.VMEM_SHARED((n_nodes, DH), jnp.float32),
            [pltpu.SemaphoreType.DMA] * nbuf,
            [pltpu.SemaphoreType.DMA] * nbuf,
            [pltpu.SemaphoreType.DMA] * 3,
        ],
    )
    def spmm(sup_hbm, src_hbm, dst_hbm, w_hbm, b_hbm, out_hbm,
             srcv, dstv, wv, bv, gbuf, sbuf, acc, gsems, ssems, stsems):
        cid = lax.axis_index("c")
        sid = lax.axis_index("s")

        pltpu.async_copy(src_hbm.at[cid, sid], srcv, stsems[0])
        pltpu.async_copy(dst_hbm.at[sid], dstv, stsems[1])
        pltpu.async_copy(w_hbm.at[sid], wv, stsems[2])

        pltpu.sync_copy(b_hbm.at[cid], bv)
        bvec = [bv[0, pl.ds(j * L, L)] for j in range(nj)]

        def brow(i, carry):
            for j in range(nj):
                sbuf[i, pl.ds(j * L, L)] = bvec[j]
            return carry

        lax.fori_loop(0, nbuf * k, brow, 0)

        pltpu.make_async_copy(src_hbm.at[cid, sid], srcv, stsems[0]).wait()

        base_r = sid * rpt
        my_rows = n_nodes - (NS - 1) * rpt
        for sel, cnt in ((sid < NS - 1, rpt), (sid == NS - 1, my_rows)):
            @pl.when(sel)
            def _():
                done = 0
                while done < cnt:
                    step = min(nbuf * k, cnt - done)
                    pltpu.sync_copy(sbuf.at[pl.ds(0, step)],
                                    acc.at[pl.ds(base_r + done, step)])
                    done += step
        pltpu.make_async_copy(dst_hbm.at[sid], dstv, stsems[1]).wait()
        pltpu.make_async_copy(w_hbm.at[sid], wv, stsems[2]).wait()

        def gslice(b):
            return gbuf.at[pl.ds(b * k, k)]

        def sslice(b):
            return sbuf.at[pl.ds(b * k, k)]

        def start_gather(c, b):
            pltpu.async_copy(sup_hbm.at[srcv.at[c]], gslice(b), gsems[b])

        def wait_gather(c, b):
            pltpu.make_async_copy(sup_hbm.at[srcv.at[c]], gslice(b),
                                  gsems[b]).wait()

        def start_scatter(c, b):
            pltpu.async_copy(sslice(b), acc.at[dstv.at[c]], ssems[b], add=True)

        def wait_scatter(c, b):
            pltpu.make_async_copy(sslice(b), acc.at[dstv.at[c]],
                                  ssems[b]).wait()

        def process(c, b):
            @plsc.parallel_loop(0, k // L, 1, unroll=4)
            def _(m):
                base_i = m * L
                w16 = wv[0, pl.ds(c * k + base_i, L)]
                for t in range(L):
                    wb = jnp.broadcast_to(w16[t], (L,))
                    i = base_i + t
                    for j in range(DH // (2 * L)):
                        v = gbuf[b * k + i, pl.ds(j * 2 * L, 2 * L)]
                        ev, od = plsc.unpack(
                            v, format=plsc.PackFormat.INTERLEAVED,
                            preferred_element_type=jnp.float32)
                        sbuf[b * k + i, pl.ds(2 * j * L, L)] = ev * wb
                        sbuf[b * k + i, pl.ds((2 * j + 1) * L, L)] = od * wb

        def slot(c, b, head, tail):
            if not head:
                wait_scatter(c - 2, (b + 2) % nbuf)
            if not tail:
                start_gather(c + 2, (b + 2) % nbuf)
            wait_gather(c, b)
            process(c, b)
            start_scatter(c, b)

        start_gather(0, 0)
        start_gather(1, 1)
        plsc.subcore_barrier()
        slot(0, 0, True, False)
        slot(1, 1, True, False)

        def quad(g, carry):
            c0 = 4 * g + 2
            for j in range(4):
                slot(c0 + j, (2 + j) % nbuf, False, False)
            return carry

        lax.fori_loop(0, (nchunk - 4) // 4, quad, 0)
        slot(nchunk - 2, (nchunk - 2) % nbuf, False, True)
        slot(nchunk - 1, (nchunk - 1) % nbuf, False, True)
        wait_scatter(nchunk - 2, (nchunk - 2) % nbuf)
        wait_scatter(nchunk - 1, (nchunk - 1) % nbuf)

        plsc.subcore_barrier()
        for sel, cnt in ((sid < NS - 1, rpt), (sid == NS - 1, my_rows)):
            @pl.when(sel)
            def _():
                pltpu.sync_copy(acc.at[pl.ds(base_r, cnt)],
                                out_hbm.at[cid, pl.ds(base_r, cnt)])

    return spmm(sup2d, src4d, dst3d, w3d, b2d)


def _split_cols(res, o_ref):
    rb = res.astype(jnp.bfloat16)
    o_ref[0] = rb[:, :DH]
    o_ref[1] = rb[:, DH:]


def _mm_first(x, W, n_nodes, blk):
    def body(x_ref, w_ref, o_ref):
        _split_cols(jnp.dot(x_ref[...], w_ref[...],
                            preferred_element_type=jnp.float32), o_ref)

    return pl.pallas_call(
        body,
        grid=(n_nodes // blk,),
        in_specs=[pl.BlockSpec((blk, 128), lambda i: (i, 0)),
                  pl.BlockSpec((128, 128), lambda i: (0, 0))],
        out_specs=pl.BlockSpec((2, blk, DH), lambda i: (0, i, 0)),
        out_shape=jax.ShapeDtypeStruct((2, n_nodes, DH), jnp.bfloat16),
    )(x, W)


def _mm_fused(parts, W, n_nodes, blk):
    def body(p_ref, w_ref, o_ref):
        h = jnp.maximum(jnp.concatenate([p_ref[0], p_ref[1]], axis=1), 0.0)
        _split_cols(jnp.dot(h, w_ref[...], preferred_element_type=jnp.float32),
                    o_ref)

    return pl.pallas_call(
        body,
        grid=(n_nodes // blk,),
        in_specs=[pl.BlockSpec((2, blk, DH), lambda i: (0, i, 0)),
                  pl.BlockSpec((128, 128), lambda i: (0, 0))],
        out_specs=pl.BlockSpec((2, blk, DH), lambda i: (0, i, 0)),
        out_shape=jax.ShapeDtypeStruct((2, n_nodes, DH), jnp.bfloat16),
    )(parts, W)


def kernel(x, edge_index, edge_weight, W1, b1, W2, b2, W3, b3, W4, b4):
    n_nodes = x.shape[0]
    e = edge_weight.shape[0]
    k = 64
    blk = 1000

    epw = e // NS
    epw_pad = -(-epw // (4 * k)) * (4 * k)
    pad = epw_pad - epw

    def worker_layout(a, fill):
        a = a.reshape(NS, epw)
        if pad:
            a = jnp.pad(a, ((0, 0), (0, pad)), constant_values=fill)
        return a

    nchunk = epw_pad // k
    dst3d = worker_layout(edge_index[0], 0).reshape(NS, nchunk, k)
    src3d = worker_layout(edge_index[1], 0).reshape(NS, nchunk, k)
    src4d = src3d[None] + (jnp.arange(NC, dtype=jnp.int32) * n_nodes)[:, None, None, None]
    w3d = worker_layout(edge_weight, 0.0).reshape(NS, 1, epw_pad)

    W4p = jnp.pad(W4, ((0, 0), (0, 128 - W4.shape[1])))
    b4p = jnp.pad(b4, (0, 128 - b4.shape[0]))
    W1p, W2p, W3p, W4p = (W[:, _COLPERM] for W in (W1, W2, W3, W4p))

    spmm = functools.partial(_spmm_sc, n_nodes=n_nodes, epw=epw_pad, k=k)

    s = _mm_first(x, W1p, n_nodes, blk)
    p = spmm(s.reshape(NC * n_nodes, DH), src4d, dst3d, w3d, b1.reshape(NC, 1, DH))
    s = _mm_fused(p, W2p, n_nodes, blk)
    p = spmm(s.reshape(NC * n_nodes, DH), src4d, dst3d, w3d, b2.reshape(NC, 1, DH))
    s = _mm_fused(p, W3p, n_nodes, blk)
    p = spmm(s.reshape(NC * n_nodes, DH), src4d, dst3d, w3d, b3.reshape(NC, 1, DH))
    s = _mm_fused(p, W4p, n_nodes, blk)
    p = spmm(s.reshape(NC * n_nodes, DH), src4d, dst3d, w3d, b4p.reshape(NC, 1, DH))
    out = jnp.concatenate([p[0], p[1]], axis=1)
    return out[:, :W4.shape[1]]
